# baseline (speedup 1.0000x reference)
import functools

import jax
import jax.numpy as jnp
from jax.experimental import pallas as pl

BATCH = 4096
INPUT_DIM = 2048
LATENT_DIM = 256
NUM_EMB = 1024
BLOCK_B = 256
COMMIT = 0.25


def _dot_t(a, w):
    return jax.lax.dot_general(a.astype(jnp.bfloat16), w,
                               (((1,), (1,)), ((), ())),
                               preferred_element_type=jnp.float32)


def _vqvae_kernel(x_ref, We1, be1, We2, be2, We3, be3, We4, be4, cb,
                  Wd1, bd1, Wd2, bd2, Wd3, bd3, Wd4, bd4,
                  z_ref, zq_ref, pred_ref, loss_ref, idx_ref):
    i = pl.program_id(0)

    h = jnp.maximum(_dot_t(x_ref[...], We1[...]) + be1[...], 0.0)
    h = jnp.maximum(_dot_t(h, We2[...]) + be2[...], 0.0)
    h = jnp.maximum(_dot_t(h, We3[...]) + be3[...], 0.0)
    z = _dot_t(h, We4[...]) + be4[...]

    c = cb[...]
    c_bf = c.astype(jnp.bfloat16)
    zsq = jnp.sum(z * z, axis=1, keepdims=True)
    csq = jnp.sum(c * c, axis=1, keepdims=True).reshape(1, NUM_EMB)
    zc = _dot_t(z, c_bf)
    dist = (zsq + csq) - 2.0 * zc

    dmin = jnp.min(dist, axis=1, keepdims=True)
    lane = jax.lax.broadcasted_iota(jnp.int32, dist.shape, 1)
    masked = jnp.where(dist == dmin, lane, NUM_EMB)
    idx = jnp.min(masked, axis=1, keepdims=True)
    idx_ref[...] = idx

    onehot = (lane == idx).astype(jnp.float32)
    zq = jax.lax.dot_general(onehot, c, (((1,), (0,)), ((), ())),
                             precision=jax.lax.Precision.HIGHEST,
                             preferred_element_type=jnp.float32)

    diff = zq - z
    partial = jnp.sum(diff * diff).reshape(1, 1)

    @pl.when(i == 0)
    def _():
        loss_ref[...] = jnp.zeros_like(loss_ref)

    loss_ref[...] += partial

    z_ref[...] = z
    zq_st = z + (zq - z)
    zq_ref[...] = zq_st

    d = jnp.maximum(_dot_t(zq_st, Wd1[...]) + bd1[...], 0.0)
    d = jnp.maximum(_dot_t(d, Wd2[...]) + bd2[...], 0.0)
    d = jnp.maximum(_dot_t(d, Wd3[...]) + bd3[...], 0.0)
    pred_ref[...] = _dot_t(d, Wd4[...]) + bd4[...]


def _full(shape):
    return pl.BlockSpec(shape, lambda i: (0,) * len(shape))


@functools.partial(jax.jit, static_argnames=())
def kernel(x, We1, be1, We2, be2, We3, be3, We4, be4, codebook,
           Wd1, bd1, Wd2, bd2, Wd3, bd3, Wd4, bd4):
    nb = BATCH // BLOCK_B
    grid = (nb,)
    b2 = lambda b: b.reshape(1, -1)
    bf = lambda w: w.astype(jnp.bfloat16)

    in_specs = [
        pl.BlockSpec((BLOCK_B, INPUT_DIM), lambda i: (i, 0)),
        _full(We1.shape), _full((1, be1.shape[0])),
        _full(We2.shape), _full((1, be2.shape[0])),
        _full(We3.shape), _full((1, be3.shape[0])),
        _full(We4.shape), _full((1, be4.shape[0])),
        _full(codebook.shape),
        _full(Wd1.shape), _full((1, bd1.shape[0])),
        _full(Wd2.shape), _full((1, bd2.shape[0])),
        _full(Wd3.shape), _full((1, bd3.shape[0])),
        _full(Wd4.shape), _full((1, bd4.shape[0])),
    ]
    out_specs = [
        pl.BlockSpec((BLOCK_B, LATENT_DIM), lambda i: (i, 0)),
        pl.BlockSpec((BLOCK_B, LATENT_DIM), lambda i: (i, 0)),
        pl.BlockSpec((BLOCK_B, INPUT_DIM), lambda i: (i, 0)),
        pl.BlockSpec((1, 1), lambda i: (0, 0)),
        pl.BlockSpec((BLOCK_B, 1), lambda i: (i, 0)),
    ]
    out_shapes = [
        jax.ShapeDtypeStruct((BATCH, LATENT_DIM), jnp.float32),
        jax.ShapeDtypeStruct((BATCH, LATENT_DIM), jnp.float32),
        jax.ShapeDtypeStruct((BATCH, INPUT_DIM), jnp.float32),
        jax.ShapeDtypeStruct((1, 1), jnp.float32),
        jax.ShapeDtypeStruct((BATCH, 1), jnp.int32),
    ]

    z, zq, pred, loss_acc, idx = pl.pallas_call(
        _vqvae_kernel,
        grid=grid,
        in_specs=in_specs,
        out_specs=out_specs,
        out_shape=out_shapes,
    )(bf(x), bf(We1), b2(be1), bf(We2), b2(be2), bf(We3), b2(be3),
      bf(We4), b2(be4), codebook,
      bf(Wd1), b2(bd1), bf(Wd2), b2(bd2), bf(Wd3), b2(bd3), bf(Wd4), b2(bd4))

    n = BATCH * LATENT_DIM
    mse = loss_acc[0, 0] / n
    loss = mse + COMMIT * mse
    return (z, zq, pred, loss, idx.reshape(BATCH))


# speedup vs baseline: 1.2191x; 1.2191x over previous
import functools

import jax
import jax.numpy as jnp
from jax.experimental import pallas as pl
from jax.experimental.pallas import tpu as pltpu

BATCH = 4096
INPUT_DIM = 2048
LATENT_DIM = 256
NUM_EMB = 1024
BLOCK_B = 512
COMMIT = 0.25


def _dot_t(a, w):
    return jax.lax.dot_general(a.astype(jnp.bfloat16), w,
                               (((1,), (1,)), ((), ())),
                               preferred_element_type=jnp.float32)


def _dot(a, w):
    return jax.lax.dot_general(a, w, (((1,), (0,)), ((), ())),
                               preferred_element_type=jnp.float32)


def _vqvae_kernel(x_ref, We1, be1, We2, be2, We3, be3, We4, be4,
                  cb, cb_hi, cb_lo,
                  Wd1, bd1, Wd2, bd2, Wd3, bd3, Wd4, bd4,
                  z_ref, zq_ref, pred_ref, loss_ref, idx_ref,
                  csq_ref):
    i = pl.program_id(0)

    h = jnp.maximum(_dot_t(x_ref[...], We1[...]) + be1[...], 0.0)
    h = jnp.maximum(_dot_t(h, We2[...]) + be2[...], 0.0)
    h = jnp.maximum(_dot_t(h, We3[...]) + be3[...], 0.0)
    z = _dot_t(h, We4[...]) + be4[...]

    @pl.when(i == 0)
    def _():
        c = cb[...]
        csq_ref[...] = jnp.sum(c * c, axis=1, keepdims=True).reshape(1, NUM_EMB)

    zsq = jnp.sum(z * z, axis=1, keepdims=True)
    zc = _dot_t(z, cb_hi[...])
    dist = (zsq + csq_ref[...]) - 2.0 * zc

    dmin = jnp.min(dist, axis=1, keepdims=True)
    lane = jax.lax.broadcasted_iota(jnp.int32, dist.shape, 1)
    masked = jnp.where(dist == dmin, lane, NUM_EMB)
    idx = jnp.min(masked, axis=1, keepdims=True)
    idx_ref[...] = idx

    onehot = (lane == idx).astype(jnp.bfloat16)
    zq = _dot(onehot, cb_hi[...]) + _dot(onehot, cb_lo[...])

    diff = zq - z
    partial = jnp.sum(diff * diff).reshape(1, 1)

    @pl.when(i == 0)
    def _():
        loss_ref[...] = jnp.zeros_like(loss_ref)

    loss_ref[...] += partial

    z_ref[...] = z
    zq_st = z + (zq - z)
    zq_ref[...] = zq_st

    d = jnp.maximum(_dot_t(zq_st, Wd1[...]) + bd1[...], 0.0)
    d = jnp.maximum(_dot_t(d, Wd2[...]) + bd2[...], 0.0)
    d = jnp.maximum(_dot_t(d, Wd3[...]) + bd3[...], 0.0)
    pred_ref[...] = _dot_t(d, Wd4[...]) + bd4[...]


def _full(shape):
    return pl.BlockSpec(shape, lambda i: (0,) * len(shape))


@functools.partial(jax.jit, static_argnames=())
def kernel(x, We1, be1, We2, be2, We3, be3, We4, be4, codebook,
           Wd1, bd1, Wd2, bd2, Wd3, bd3, Wd4, bd4):
    nb = BATCH // BLOCK_B
    grid = (nb,)
    b2 = lambda b: b.reshape(1, -1)
    bf = lambda w: w.astype(jnp.bfloat16)

    cb_hi = codebook.astype(jnp.bfloat16)
    cb_lo = (codebook - cb_hi.astype(jnp.float32)).astype(jnp.bfloat16)

    in_specs = [
        pl.BlockSpec((BLOCK_B, INPUT_DIM), lambda i: (i, 0)),
        _full(We1.shape), _full((1, be1.shape[0])),
        _full(We2.shape), _full((1, be2.shape[0])),
        _full(We3.shape), _full((1, be3.shape[0])),
        _full(We4.shape), _full((1, be4.shape[0])),
        _full(codebook.shape), _full(codebook.shape), _full(codebook.shape),
        _full(Wd1.shape), _full((1, bd1.shape[0])),
        _full(Wd2.shape), _full((1, bd2.shape[0])),
        _full(Wd3.shape), _full((1, bd3.shape[0])),
        _full(Wd4.shape), _full((1, bd4.shape[0])),
    ]
    out_specs = [
        pl.BlockSpec((BLOCK_B, LATENT_DIM), lambda i: (i, 0)),
        pl.BlockSpec((BLOCK_B, LATENT_DIM), lambda i: (i, 0)),
        pl.BlockSpec((BLOCK_B, INPUT_DIM), lambda i: (i, 0)),
        pl.BlockSpec((1, 1), lambda i: (0, 0)),
        pl.BlockSpec((BLOCK_B, 1), lambda i: (i, 0)),
    ]
    out_shapes = [
        jax.ShapeDtypeStruct((BATCH, LATENT_DIM), jnp.float32),
        jax.ShapeDtypeStruct((BATCH, LATENT_DIM), jnp.float32),
        jax.ShapeDtypeStruct((BATCH, INPUT_DIM), jnp.float32),
        jax.ShapeDtypeStruct((1, 1), jnp.float32),
        jax.ShapeDtypeStruct((BATCH, 1), jnp.int32),
    ]

    z, zq, pred, loss_acc, idx = pl.pallas_call(
        _vqvae_kernel,
        grid=grid,
        in_specs=in_specs,
        out_specs=out_specs,
        out_shape=out_shapes,
        scratch_shapes=[pltpu.VMEM((1, NUM_EMB), jnp.float32)],
    )(bf(x), bf(We1), b2(be1), bf(We2), b2(be2), bf(We3), b2(be3),
      bf(We4), b2(be4), codebook, cb_hi, cb_lo,
      bf(Wd1), b2(bd1), bf(Wd2), b2(bd2), bf(Wd3), b2(bd3), bf(Wd4), b2(bd4))

    n = BATCH * LATENT_DIM
    mse = loss_acc[0, 0] / n
    loss = mse + COMMIT * mse
    return (z, zq, pred, loss, idx.reshape(BATCH))


# speedup vs baseline: 1.4977x; 1.2285x over previous
import functools

import jax
import jax.numpy as jnp
from jax.experimental import pallas as pl
from jax.experimental.pallas import tpu as pltpu

BATCH = 4096
INPUT_DIM = 2048
LATENT_DIM = 256
NUM_EMB = 1024
BLOCK_B = 512
COMMIT = 0.25

_BF = jnp.bfloat16


def _dot_t(a, w):
    return jax.lax.dot_general(a.astype(_BF), w, (((1,), (1,)), ((), ())),
                               preferred_element_type=jnp.float32)


def _dot(a, w):
    return jax.lax.dot_general(a, w, (((1,), (0,)), ((), ())),
                               preferred_element_type=jnp.float32)


def _enc_kernel(x_ref, We1, be1, We2, be2, We3, be3, We4, be4, cb,
                z_ref, zq_ref, loss_ref, idx_ref,
                w1s, w2s, w3s, w4s, chs, cls, csq_ref):
    i = pl.program_id(0)

    @pl.when(i == 0)
    def _():
        w1s[...] = We1[...].astype(_BF)
        w2s[...] = We2[...].astype(_BF)
        w3s[...] = We3[...].astype(_BF)
        w4s[...] = We4[...].astype(_BF)
        c = cb[...]
        hi = c.astype(_BF)
        chs[...] = hi
        cls[...] = (c - hi.astype(jnp.float32)).astype(_BF)
        csq_ref[...] = jnp.sum(c * c, axis=1, keepdims=True).reshape(1, NUM_EMB)
        loss_ref[...] = jnp.zeros_like(loss_ref)

    h = jnp.maximum(_dot_t(x_ref[...], w1s[...]) + be1[...], 0.0)
    h = jnp.maximum(_dot_t(h, w2s[...]) + be2[...], 0.0)
    h = jnp.maximum(_dot_t(h, w3s[...]) + be3[...], 0.0)
    z = _dot_t(h, w4s[...]) + be4[...]

    zsq = jnp.sum(z * z, axis=1, keepdims=True)
    zc = _dot_t(z, chs[...])
    dist = (zsq + csq_ref[...]) - 2.0 * zc

    dmin = jnp.min(dist, axis=1, keepdims=True)
    lane = jax.lax.broadcasted_iota(jnp.int32, dist.shape, 1)
    masked = jnp.where(dist == dmin, lane, NUM_EMB)
    idx = jnp.min(masked, axis=1, keepdims=True)
    idx_ref[...] = idx

    onehot = (lane == idx).astype(_BF)
    zq = _dot(onehot, chs[...]) + _dot(onehot, cls[...])

    diff = zq - z
    loss_ref[...] += jnp.sum(diff * diff).reshape(1, 1)

    z_ref[...] = z
    zq_ref[...] = z + (zq - z)


def _dec_kernel(zq_ref, Wd1, bd1, Wd2, bd2, Wd3, bd3, Wd4, bd4,
                pred_ref, w1s, w2s, w3s, w4s):
    i = pl.program_id(0)

    @pl.when(i == 0)
    def _():
        w1s[...] = Wd1[...].astype(_BF)
        w2s[...] = Wd2[...].astype(_BF)
        w3s[...] = Wd3[...].astype(_BF)
        w4s[...] = Wd4[...].astype(_BF)

    d = jnp.maximum(_dot_t(zq_ref[...], w1s[...]) + bd1[...], 0.0)
    d = jnp.maximum(_dot_t(d, w2s[...]) + bd2[...], 0.0)
    d = jnp.maximum(_dot_t(d, w3s[...]) + bd3[...], 0.0)
    pred_ref[...] = _dot_t(d, w4s[...]) + bd4[...]


def _full(shape):
    return pl.BlockSpec(shape, lambda i: (0,) * len(shape))


@functools.partial(jax.jit, static_argnames=())
def kernel(x, We1, be1, We2, be2, We3, be3, We4, be4, codebook,
           Wd1, bd1, Wd2, bd2, Wd3, bd3, Wd4, bd4):
    nb = BATCH // BLOCK_B
    b2 = lambda b: b.reshape(1, -1)
    vmem_bf = lambda shape: pltpu.VMEM(shape, _BF)

    z, zq, loss_acc, idx = pl.pallas_call(
        _enc_kernel,
        grid=(nb,),
        in_specs=[
            pl.BlockSpec((BLOCK_B, INPUT_DIM), lambda i: (i, 0)),
            _full(We1.shape), _full((1, be1.shape[0])),
            _full(We2.shape), _full((1, be2.shape[0])),
            _full(We3.shape), _full((1, be3.shape[0])),
            _full(We4.shape), _full((1, be4.shape[0])),
            _full(codebook.shape),
        ],
        out_specs=[
            pl.BlockSpec((BLOCK_B, LATENT_DIM), lambda i: (i, 0)),
            pl.BlockSpec((BLOCK_B, LATENT_DIM), lambda i: (i, 0)),
            pl.BlockSpec((1, 1), lambda i: (0, 0)),
            pl.BlockSpec((BLOCK_B, 1), lambda i: (i, 0)),
        ],
        out_shape=[
            jax.ShapeDtypeStruct((BATCH, LATENT_DIM), jnp.float32),
            jax.ShapeDtypeStruct((BATCH, LATENT_DIM), jnp.float32),
            jax.ShapeDtypeStruct((1, 1), jnp.float32),
            jax.ShapeDtypeStruct((BATCH, 1), jnp.int32),
        ],
        scratch_shapes=[
            vmem_bf(We1.shape), vmem_bf(We2.shape),
            vmem_bf(We3.shape), vmem_bf(We4.shape),
            vmem_bf(codebook.shape), vmem_bf(codebook.shape),
            pltpu.VMEM((1, NUM_EMB), jnp.float32),
        ],
    )(x, We1, b2(be1), We2, b2(be2), We3, b2(be3), We4, b2(be4), codebook)

    pred = pl.pallas_call(
        _dec_kernel,
        grid=(nb,),
        in_specs=[
            pl.BlockSpec((BLOCK_B, LATENT_DIM), lambda i: (i, 0)),
            _full(Wd1.shape), _full((1, bd1.shape[0])),
            _full(Wd2.shape), _full((1, bd2.shape[0])),
            _full(Wd3.shape), _full((1, bd3.shape[0])),
            _full(Wd4.shape), _full((1, bd4.shape[0])),
        ],
        out_specs=pl.BlockSpec((BLOCK_B, INPUT_DIM), lambda i: (i, 0)),
        out_shape=jax.ShapeDtypeStruct((BATCH, INPUT_DIM), jnp.float32),
        scratch_shapes=[
            vmem_bf(Wd1.shape), vmem_bf(Wd2.shape),
            vmem_bf(Wd3.shape), vmem_bf(Wd4.shape),
        ],
    )(zq, Wd1, b2(bd1), Wd2, b2(bd2), Wd3, b2(bd3), Wd4, b2(bd4))

    n = BATCH * LATENT_DIM
    mse = loss_acc[0, 0] / n
    loss = mse + COMMIT * mse
    return (z, zq, pred, loss, idx.reshape(BATCH))
